# baseline (speedup 1.0000x reference)
import functools

import jax
import jax.numpy as jnp
from jax import lax
from jax.experimental import pallas as pl
from jax.experimental.pallas import tpu as pltpu
from jax.experimental.pallas import tpu_sc as plsc

D = 64
DP = 128
NC, NS = 2, 16
NW = NC * NS
SPLIT = 128
NBUF = 4
NB2 = 2 * NBUF
TBLK = 8192


def _tc_repack(tt):
    F, V = tt.shape
    grid = (V + TBLK - 1) // TBLK

    def body(tt_ref, out_ref):
        half = TBLK // 2
        xa = tt_ref[:, pl.ds(0, half)]
        xb = tt_ref[:, pl.ds(half, half)]
        ya = xa.T
        eye = jnp.eye(F, dtype=jnp.float32)
        yb = lax.dot_general(
            xb, eye, (((0,), (0,)), ((), ())),
            preferred_element_type=jnp.float32,
            precision=lax.Precision.HIGHEST,
        )
        out_ref[pl.ds(0, half), :] = jnp.concatenate([ya, ya], axis=1)
        out_ref[pl.ds(half, half), :] = jnp.concatenate([yb, yb], axis=1)

    return pl.pallas_call(
        body,
        grid=(grid,),
        in_specs=[pl.BlockSpec((F, TBLK), lambda i: (0, i))],
        out_specs=pl.BlockSpec((TBLK, 2 * F), lambda i: (i, 0)),
        out_shape=jax.ShapeDtypeStruct((V, 2 * F), jnp.float32),
    )(tt)


def _sc_embed(idx, table):
    B, H = idx.shape
    assert B % NW == 0 and table.shape[1] == D
    R = B // NW
    assert (R - 2 * NBUF) % NB2 == 0 and R >= 2 * NB2
    mesh = plsc.VectorSubcoreMesh(core_axis_name="c", subcore_axis_name="s")

    @functools.partial(
        pl.kernel,
        out_type=jax.ShapeDtypeStruct((B, H, DP), jnp.float32),
        mesh=mesh,
        compiler_params=pltpu.CompilerParams(use_tc_tiling_on_sc=False),
        scratch_types=[
            pltpu.VMEM((R, H), jnp.int32),
            [pltpu.VMEM((1, H, D), jnp.float32) for _ in range(NB2)],
            [pltpu.SemaphoreType.DMA for _ in range(NB2)],
            [pltpu.SemaphoreType.DMA for _ in range(NB2)],
        ],
    )
    def k(table_hbm, idx_hbm, out_hbm, idx_v, bufs, gsems, osems):
        wid = lax.axis_index("s") * NC + lax.axis_index("c")
        rbase = wid * R
        pltpu.sync_copy(idx_hbm.at[pl.ds(rbase, R)], idx_v)

        def gathers(r, b):
            return (
                pltpu.make_async_copy(
                    table_hbm.at[idx_v.at[r, pl.ds(0, SPLIT)]],
                    bufs[b].at[0, pl.ds(0, SPLIT)],
                    gsems[b],
                ),
                pltpu.make_async_copy(
                    table_hbm.at[idx_v.at[r, pl.ds(SPLIT, H - SPLIT)]],
                    bufs[b].at[0, pl.ds(SPLIT, H - SPLIT)],
                    gsems[b],
                ),
            )

        def fire(r, b):
            g0, g1 = gathers(r, b)
            g0.start()
            g1.start()

        def drain(r, b):
            g0, g1 = gathers(r, b)
            g0.wait()
            g1.wait()

        def write(r, b):
            return pltpu.make_async_copy(
                bufs[b],
                out_hbm.at[pl.ds(rbase + r, 1), pl.ds(0, H), pl.ds(0, D)],
                osems[b],
            )

        for b in range(NBUF):
            fire(b, b)

        for r in range(NBUF):
            drain(r, r)
            write(r, r).start()
            fire(r + NBUF, r + NBUF)

        def body(o, carry):
            for s in range(NB2):
                r = NBUF + o * NB2 + s
                b = (NBUF + s) % NB2
                drain(r, b)
                write(r, b).start()
                j = r + NBUF
                bj = s
                write(j - NB2, bj).wait()
                fire(j, bj)
            return carry

        lax.fori_loop(0, (R - 2 * NBUF) // NB2, body, 0)

        for r in range(R - NBUF, R):
            b = r % NB2
            drain(r, b)
            write(r, b).start()

        for b in range(NB2):
            write(R - NB2 + b, b).wait()

    return k(table, idx)


def kernel(inputs, table):
    wide = _tc_repack(table.T)
    compact = wide.reshape(-1, D)
    out = _sc_embed(inputs.astype(jnp.int32) * 2, compact)
    return out[:, :, :D]


# speedup vs baseline: 1.0673x; 1.0673x over previous
import functools

import jax
import jax.numpy as jnp
from jax import lax
from jax.experimental import pallas as pl
from jax.experimental.pallas import tpu as pltpu
from jax.experimental.pallas import tpu_sc as plsc

D = 64
DP = 128
NC, NS = 2, 16
NW = NC * NS
SPLIT = 128
NBUF = 4
NB2 = 2 * NBUF
TBLK = 16384


def _tc_repack(tt):
    F, V = tt.shape
    grid = (V + TBLK - 1) // TBLK

    def body(tt_ref, out_ref):
        y = tt_ref[...].T
        out_ref[:, pl.ds(0, F)] = y
        out_ref[:, pl.ds(F, F)] = y

    return pl.pallas_call(
        body,
        grid=(grid,),
        in_specs=[pl.BlockSpec((F, TBLK), lambda i: (0, i))],
        out_specs=pl.BlockSpec((TBLK, 2 * F), lambda i: (i, 0)),
        out_shape=jax.ShapeDtypeStruct((V, 2 * F), jnp.float32),
    )(tt)


def _sc_embed(idx, table):
    B, H = idx.shape
    assert B % NW == 0 and table.shape[1] == D
    R = B // NW
    assert (R - 2 * NBUF) % NB2 == 0 and R >= 2 * NB2
    mesh = plsc.VectorSubcoreMesh(core_axis_name="c", subcore_axis_name="s")

    @functools.partial(
        pl.kernel,
        out_type=jax.ShapeDtypeStruct((B, H, DP), jnp.float32),
        mesh=mesh,
        compiler_params=pltpu.CompilerParams(use_tc_tiling_on_sc=False),
        scratch_types=[
            pltpu.VMEM((R, H), jnp.int32),
            [pltpu.VMEM((1, H, D), jnp.float32) for _ in range(NB2)],
            [pltpu.SemaphoreType.DMA for _ in range(NB2)],
            [pltpu.SemaphoreType.DMA for _ in range(NB2)],
        ],
    )
    def k(table_hbm, idx_hbm, out_hbm, idx_v, bufs, gsems, osems):
        wid = lax.axis_index("s") * NC + lax.axis_index("c")
        rbase = wid * R
        pltpu.sync_copy(idx_hbm.at[pl.ds(rbase, R)], idx_v)

        def gathers(r, b):
            return (
                pltpu.make_async_copy(
                    table_hbm.at[idx_v.at[r, pl.ds(0, SPLIT)]],
                    bufs[b].at[0, pl.ds(0, SPLIT)],
                    gsems[b],
                ),
                pltpu.make_async_copy(
                    table_hbm.at[idx_v.at[r, pl.ds(SPLIT, H - SPLIT)]],
                    bufs[b].at[0, pl.ds(SPLIT, H - SPLIT)],
                    gsems[b],
                ),
            )

        def fire(r, b):
            g0, g1 = gathers(r, b)
            g0.start()
            g1.start()

        def drain(r, b):
            g0, g1 = gathers(r, b)
            g0.wait()
            g1.wait()

        def write(r, b):
            return pltpu.make_async_copy(
                bufs[b],
                out_hbm.at[pl.ds(rbase + r, 1), pl.ds(0, H), pl.ds(0, D)],
                osems[b],
            )

        for b in range(NBUF):
            fire(b, b)

        for r in range(NBUF):
            drain(r, r)
            write(r, r).start()
            fire(r + NBUF, r + NBUF)

        def body(o, carry):
            for s in range(NB2):
                r = NBUF + o * NB2 + s
                b = (NBUF + s) % NB2
                drain(r, b)
                write(r, b).start()
                j = r + NBUF
                bj = s
                write(j - NB2, bj).wait()
                fire(j, bj)
            return carry

        lax.fori_loop(0, (R - 2 * NBUF) // NB2, body, 0)

        for r in range(R - NBUF, R):
            b = r % NB2
            drain(r, b)
            write(r, b).start()

        for b in range(NB2):
            write(R - NB2 + b, b).wait()

    return k(table, idx)


def kernel(inputs, table):
    wide = _tc_repack(table.T)
    compact = wide.reshape(-1, D)
    out = _sc_embed(inputs.astype(jnp.int32) * 2, compact)
    return out[:, :, :D]


# speedup vs baseline: 1.0841x; 1.0158x over previous
import functools

import jax
import jax.numpy as jnp
from jax import lax
from jax.experimental import pallas as pl
from jax.experimental.pallas import tpu as pltpu
from jax.experimental.pallas import tpu_sc as plsc

D = 64
DP = 128
NC, NS = 2, 16
NW = NC * NS
SPLIT = 128
NBUF = 4
NB2 = 2 * NBUF
TBLK = 32768


def _tc_repack(tt):
    F, V = tt.shape
    grid = (V + TBLK - 1) // TBLK

    def body(tt_ref, out_ref):
        y = tt_ref[...].T
        out_ref[:, pl.ds(0, F)] = y
        out_ref[:, pl.ds(F, F)] = y

    return pl.pallas_call(
        body,
        grid=(grid,),
        in_specs=[pl.BlockSpec((F, TBLK), lambda i: (0, i))],
        out_specs=pl.BlockSpec((TBLK, 2 * F), lambda i: (i, 0)),
        out_shape=jax.ShapeDtypeStruct((V, 2 * F), jnp.float32),
        compiler_params=pltpu.CompilerParams(
            vmem_limit_bytes=100 * 1024 * 1024
        ),
    )(tt)


def _sc_embed(idx, table):
    B, H = idx.shape
    assert B % NW == 0 and table.shape[1] == D
    R = B // NW
    assert (R - 2 * NBUF) % NB2 == 0 and R >= 2 * NB2
    mesh = plsc.VectorSubcoreMesh(core_axis_name="c", subcore_axis_name="s")

    @functools.partial(
        pl.kernel,
        out_type=jax.ShapeDtypeStruct((B, H, DP), jnp.float32),
        mesh=mesh,
        compiler_params=pltpu.CompilerParams(use_tc_tiling_on_sc=False),
        scratch_types=[
            pltpu.VMEM((R, H), jnp.int32),
            [pltpu.VMEM((1, H, D), jnp.float32) for _ in range(NB2)],
            [pltpu.SemaphoreType.DMA for _ in range(NB2)],
            [pltpu.SemaphoreType.DMA for _ in range(NB2)],
        ],
    )
    def k(table_hbm, idx_hbm, out_hbm, idx_v, bufs, gsems, osems):
        wid = lax.axis_index("s") * NC + lax.axis_index("c")
        rbase = wid * R
        pltpu.sync_copy(idx_hbm.at[pl.ds(rbase, R)], idx_v)

        def gathers(r, b):
            return (
                pltpu.make_async_copy(
                    table_hbm.at[idx_v.at[r, pl.ds(0, SPLIT)]],
                    bufs[b].at[0, pl.ds(0, SPLIT)],
                    gsems[b],
                ),
                pltpu.make_async_copy(
                    table_hbm.at[idx_v.at[r, pl.ds(SPLIT, H - SPLIT)]],
                    bufs[b].at[0, pl.ds(SPLIT, H - SPLIT)],
                    gsems[b],
                ),
            )

        def fire(r, b):
            g0, g1 = gathers(r, b)
            g0.start()
            g1.start()

        def drain(r, b):
            g0, g1 = gathers(r, b)
            g0.wait()
            g1.wait()

        def write(r, b):
            return pltpu.make_async_copy(
                bufs[b],
                out_hbm.at[pl.ds(rbase + r, 1), pl.ds(0, H), pl.ds(0, D)],
                osems[b],
            )

        for b in range(NBUF):
            fire(b, b)

        for r in range(NBUF):
            drain(r, r)
            write(r, r).start()
            fire(r + NBUF, r + NBUF)

        def body(o, carry):
            for s in range(NB2):
                r = NBUF + o * NB2 + s
                b = (NBUF + s) % NB2
                drain(r, b)
                write(r, b).start()
                j = r + NBUF
                bj = s
                write(j - NB2, bj).wait()
                fire(j, bj)
            return carry

        lax.fori_loop(0, (R - 2 * NBUF) // NB2, body, 0)

        for r in range(R - NBUF, R):
            b = r % NB2
            drain(r, b)
            write(r, b).start()

        for b in range(NB2):
            write(R - NB2 + b, b).wait()

    return k(table, idx)


def kernel(inputs, table):
    wide = _tc_repack(table.T)
    compact = wide.reshape(-1, D)
    out = _sc_embed(inputs.astype(jnp.int32) * 2, compact)
    return out[:, :, :D]
